# Initial kernel scaffold; baseline (speedup 1.0000x reference)
import jax
import jax.numpy as jnp
from jax.experimental import pallas as pl


def kernel(edge_index, edge_weight, user_emb, item_emb):
    raise NotImplementedError("write your pallas kernel here")


# speedup vs baseline: 3.0201x; 3.0201x over previous
import functools

import numpy as np

import jax
import jax.numpy as jnp
from jax import lax
from jax.experimental import pallas as pl
from jax.experimental.pallas import tpu as pltpu
from jax.experimental.pallas import tpu_sc as plsc

N_USERS = 25000
N_ITEMS = 25000
N = N_USERS + N_ITEMS
D = 64
E = 800000
NUM_LAYERS = 3

NC, NS = 2, 16
HALF = D // NC
N_PAD = 50048
ROWS_PER_TILE = N_PAD // NS
CO_CHUNK = 184
CO_STEPS = ROWS_PER_TILE // CO_CHUNK

CHUNK = 128
E_PAD = 819200
N_CHUNKS = E_PAD // CHUNK
CHUNKS_PER_TILE = N_CHUNKS // NS


def _sc_body(table0, src, dst, w, out1, out2, out3,
             acc, src_v, dst_v, w_v, rows_v, zbuf, sem):
    c = lax.axis_index("c")
    s = lax.axis_index("s")
    coff = c * N_PAD
    r0 = s * ROWS_PER_TILE

    zero16 = jnp.zeros((16,), jnp.float32)

    def _zb(i, carry):
        zbuf[i, pl.ds(0, 16)] = zero16
        zbuf[i, pl.ds(16, 16)] = zero16
        return carry

    lax.fori_loop(0, CO_CHUNK, _zb, 0)

    def _zero_acc(i, carry):
        pltpu.sync_copy(zbuf, acc.at[pl.ds(r0 + i * CO_CHUNK, CO_CHUNK)])
        return carry

    lax.fori_loop(0, CO_STEPS, _zero_acc, 0)
    plsc.subcore_barrier()

    tables_in = (table0, out1, out2)
    tables_out = (out1, out2, out3)
    for layer in range(NUM_LAYERS):
        tin = tables_in[layer]
        tout = tables_out[layer]

        def _chunk(j, carry, tin=tin):
            row = s * CHUNKS_PER_TILE + j
            pltpu.sync_copy(src.at[row], src_v)
            pltpu.sync_copy(dst.at[row], dst_v)
            pltpu.sync_copy(w.at[row], w_v)
            for q in range(CHUNK // 16):
                src_v[pl.ds(q * 16, 16)] = src_v[pl.ds(q * 16, 16)] + coff
            pltpu.async_copy(tin.at[src_v], rows_v, sem).wait()

            def _mul(t, mcarry):
                wv = w_v[pl.ds(t * 16, 16)]
                for q in range(16):
                    e = t * 16 + q
                    ws = lax.gather(
                        wv, jnp.full((16, 1), q, jnp.int32),
                        lax.GatherDimensionNumbers(
                            offset_dims=(), collapsed_slice_dims=(0,),
                            start_index_map=(0,)),
                        (1,), mode=lax.GatherScatterMode.PROMISE_IN_BOUNDS)
                    rows_v[e, pl.ds(0, 16)] = rows_v[e, pl.ds(0, 16)] * ws
                    rows_v[e, pl.ds(16, 16)] = rows_v[e, pl.ds(16, 16)] * ws
                return mcarry

            lax.fori_loop(0, CHUNK // 16, _mul, 0)
            pltpu.sync_copy(rows_v, acc.at[dst_v], add=True)
            return carry

        lax.fori_loop(0, CHUNKS_PER_TILE, _chunk, 0)
        plsc.subcore_barrier()

        def _co(i, carry, tout=tout):
            base = r0 + i * CO_CHUNK
            pltpu.sync_copy(acc.at[pl.ds(base, CO_CHUNK)],
                            tout.at[pl.ds(coff + base, CO_CHUNK)])
            pltpu.sync_copy(zbuf, acc.at[pl.ds(base, CO_CHUNK)])
            return carry

        lax.fori_loop(0, CO_STEPS, _co, 0)
        plsc.subcore_barrier()


_sc_propagate = pl.kernel(
    _sc_body,
    out_type=[jax.ShapeDtypeStruct((NC * N_PAD, HALF), jnp.float32)] * 3,
    mesh=plsc.VectorSubcoreMesh(core_axis_name="c", subcore_axis_name="s"),
    compiler_params=pltpu.CompilerParams(use_tc_tiling_on_sc=False),
    scratch_types=[
        pltpu.VMEM_SHARED((N_PAD, HALF), jnp.float32),
        pltpu.VMEM((CHUNK,), jnp.int32),
        pltpu.VMEM((CHUNK,), jnp.int32),
        pltpu.VMEM((CHUNK,), jnp.float32),
        pltpu.VMEM((CHUNK, HALF), jnp.float32),
        pltpu.VMEM((CO_CHUNK, HALF), jnp.float32),
        pltpu.SemaphoreType.DMA,
    ],
)

MROWS = NC * N_PAD * HALF // 128
MBLK = MROWS // 8


def _mean_body(a_ref, b_ref, c_ref, d_ref, o_ref):
    o_ref[...] = 0.25 * (a_ref[...] + b_ref[...] + c_ref[...] + d_ref[...])


_mean4 = pl.pallas_call(
    _mean_body,
    out_shape=jax.ShapeDtypeStruct((MROWS, 128), jnp.float32),
    grid=(MROWS // MBLK,),
    in_specs=[pl.BlockSpec((MBLK, 128), lambda i: (i, 0))] * 4,
    out_specs=pl.BlockSpec((MBLK, 128), lambda i: (i, 0)),
)


def kernel(edge_index, edge_weight, user_emb, item_emb):
    src = edge_index[1].astype(jnp.int32)
    dst = edge_index[0].astype(jnp.int32)
    pad = E_PAD - E
    src = jnp.concatenate([src, jnp.zeros((pad,), jnp.int32)]).reshape(N_CHUNKS, CHUNK)
    dst = jnp.concatenate([dst, jnp.zeros((pad,), jnp.int32)]).reshape(N_CHUNKS, CHUNK)
    w = jnp.concatenate([edge_weight.astype(jnp.float32),
                         jnp.zeros((pad,), jnp.float32)]).reshape(N_CHUNKS, CHUNK)

    all_emb = jnp.concatenate([user_emb, item_emb], axis=0)
    emb_pad = jnp.pad(all_emb, ((0, N_PAD - N), (0, 0)))
    table0 = emb_pad.reshape(N_PAD, NC, HALF).transpose(1, 0, 2)
    table0 = table0.reshape(NC * N_PAD, HALF)

    out1, out2, out3 = _sc_propagate(table0, src, dst, w)

    final = _mean4(table0.reshape(MROWS, 128), out1.reshape(MROWS, 128),
                   out2.reshape(MROWS, 128), out3.reshape(MROWS, 128))
    final = final.reshape(NC, N_PAD, HALF).transpose(1, 0, 2).reshape(N_PAD, D)
    final = final[:N]
    return (final[:N_USERS], final[N_USERS:])
